# baseline (speedup 1.0000x reference)
import functools

import jax
import jax.numpy as jnp
from jax import lax
from jax.experimental import pallas as pl
from jax.experimental.pallas import tpu as pltpu
from jax.experimental.pallas import tpu_sc as plsc

N = 32768
C = 2048
NC, NS, L = 2, 16, 16
NW = NC * NS
PER_W = N // NW
CB = 128
NCB = C // CB
SUB = 512
NSUB = PER_W // SUB
VPS = SUB // L
IGN = -1


def _sc_partials(logits, target):
    mesh = plsc.VectorSubcoreMesh(core_axis_name="c", subcore_axis_name="s")

    @functools.partial(
        pl.kernel,
        out_type=jax.ShapeDtypeStruct((NW * 2 * L,), jnp.float32),
        mesh=mesh,
        compiler_params=pltpu.CompilerParams(
            use_tc_tiling_on_sc=False, needs_layout_passes=False
        ),
        scratch_types=[
            pltpu.VMEM((PER_W,), jnp.int32),
            pltpu.VMEM((PER_W, L), jnp.float32),
            pltpu.VMEM((2 * L,), jnp.float32),
            pltpu.SemaphoreType.DMA,
        ],
    )
    def kern(logits_hbm, tgt_hbm, out_hbm, tgt_v, gat_v, acc_v, sem):
        wid = lax.axis_index("s") * NC + lax.axis_index("c")
        base = wid * PER_W

        pltpu.sync_copy(tgt_hbm.at[pl.ds(base, PER_W)], tgt_v)

        lane = lax.iota(jnp.int32, L)

        def fire(v, carry):
            t16 = tgt_v[pl.ds(v * L, L)]
            safe = jnp.where(t16 >= 0, t16, 0)
            c16 = safe >> 4
            for l in range(L):
                row = base + v * L + l
                cstart = pl.multiple_of(c16[l] << 4, L)
                pltpu.make_async_copy(
                    logits_hbm.at[row, pl.ds(cstart, L)],
                    gat_v.at[v * L + l],
                    sem,
                ).start()
            return carry

        lax.fori_loop(0, PER_W // L, fire, 0, unroll=False)

        for _ in range(16):
            pltpu.make_async_copy(
                logits_hbm.at[pl.ds(0, PER_W // 16), pl.ds(0, L)],
                gat_v.at[pl.ds(0, PER_W // 16)],
                sem,
            ).wait()

        def extract(v, carry):
            a_sq, a_m = carry
            t16 = tgt_v[pl.ds(v * L, L)]
            safe = jnp.where(t16 >= 0, t16, 0)
            m16 = jnp.where(t16 >= 0, 1.0, 0.0).astype(jnp.float32)
            slot = v * L + lane
            col = safe & (L - 1)
            g16 = plsc.load_gather(gat_v, [slot, col])
            d = 1.0 - g16
            return a_sq + d * d * m16, a_m + m16

        acc_sq, acc_m = lax.fori_loop(
            0, PER_W // L, extract,
            (jnp.zeros((L,), jnp.float32), jnp.zeros((L,), jnp.float32)),
            unroll=False,
        )

        acc_v[pl.ds(0, L)] = acc_sq
        acc_v[pl.ds(L, L)] = acc_m
        pltpu.sync_copy(acc_v.at[pl.ds(0, L)], out_hbm.at[pl.ds(wid * L, L)])
        pltpu.sync_copy(
            acc_v.at[pl.ds(L, L)], out_hbm.at[pl.ds(NW * L + wid * L, L)]
        )

    return kern(logits, target)


def _tc_finalize(partials):
    def body(p_ref, o_ref):
        p = p_ref[...]
        s = jnp.sum(p[0:4])
        m = jnp.sum(p[4:8])
        o_ref[...] = jnp.full((1, 1), s / m, jnp.float32)

    return pl.pallas_call(
        body,
        out_shape=jax.ShapeDtypeStruct((1, 1), jnp.float32),
    )(partials)


@jax.jit
def kernel(contrast_logits, contrast_target):
    partials = _sc_partials(contrast_logits, contrast_target)
    loss = _tc_finalize(partials.reshape(8, 128))
    return loss[0, 0]


# speedup vs baseline: 3.0599x; 3.0599x over previous
import functools

import jax
import jax.numpy as jnp
from jax import lax
from jax.experimental import pallas as pl
from jax.experimental.pallas import tpu as pltpu
from jax.experimental.pallas import tpu_sc as plsc

N = 32768
C = 2048
NC, NS, L = 2, 16, 16
NW = NC * NS
PER_W = N // NW
CB = 128
NCB = C // CB
TPB = 32
NBATCH = PER_W // TPB


def _sc_partials(logits, target):
    mesh = plsc.VectorSubcoreMesh(core_axis_name="c", subcore_axis_name="s")

    @functools.partial(
        pl.kernel,
        out_type=jax.ShapeDtypeStruct((NW * 2 * L,), jnp.float32),
        mesh=mesh,
        compiler_params=pltpu.CompilerParams(
            use_tc_tiling_on_sc=True, needs_layout_passes=False
        ),
        scratch_types=[
            pltpu.VMEM((PER_W,), jnp.int32),
            pltpu.VMEM((2, TPB, 8, CB), jnp.float32),
            pltpu.VMEM((2 * L,), jnp.float32),
            pltpu.SemaphoreType.DMA,
        ],
    )
    def kern(logits_hbm, tgt_hbm, out_hbm, tgt_v, gat_v, acc_v, sem):
        wid = lax.axis_index("s") * NC + lax.axis_index("c")
        base = wid * PER_W

        pltpu.sync_copy(tgt_hbm.at[pl.ds(base, PER_W)], tgt_v)

        lane = lax.iota(jnp.int32, L)

        def fire(b):
            p = b & 1
            for q in range(TPB // L):
                t16 = tgt_v[pl.ds(b * TPB + q * L, L)]
                cb16 = jnp.where(t16 >= 0, t16, 0) >> 7
                for l in range(L):
                    e = q * L + l
                    row = base + b * TPB + e
                    rowa = pl.multiple_of((row >> 3) << 3, 8)
                    cstart = pl.multiple_of(cb16[l] << 7, CB)
                    pltpu.make_async_copy(
                        logits_hbm.at[pl.ds(rowa, 8), pl.ds(cstart, CB)],
                        gat_v.at[p, e],
                        sem,
                    ).start()

        def drain():
            for _ in range(4):
                pltpu.make_async_copy(
                    logits_hbm.at[pl.ds(0, 8 * (TPB // 4)), pl.ds(0, CB)],
                    gat_v.at[0, pl.ds(0, TPB // 4)],
                    sem,
                ).wait()

        def extract(b, acc):
            a_sq, a_m = acc
            p = b & 1
            for q in range(TPB // L):
                t16 = tgt_v[pl.ds(b * TPB + q * L, L)]
                safe = jnp.where(t16 >= 0, t16, 0)
                m16 = jnp.where(t16 >= 0, 1.0, 0.0).astype(jnp.float32)
                slot = q * L + lane
                sub = (base + b * TPB + slot) & 7
                col = safe & (CB - 1)
                g16 = plsc.load_gather(gat_v.at[p], [slot, sub, col])
                d = 1.0 - g16
                a_sq = a_sq + d * d * m16
                a_m = a_m + m16
            return a_sq, a_m

        fire(0)

        def body(b, acc):
            fire(b + 1)
            drain()
            return extract(b, acc)

        acc_sq, acc_m = lax.fori_loop(
            0, NBATCH - 1, body,
            (jnp.zeros((L,), jnp.float32), jnp.zeros((L,), jnp.float32)),
            unroll=False,
        )
        drain()
        acc_sq, acc_m = extract(NBATCH - 1, (acc_sq, acc_m))

        acc_v[pl.ds(0, L)] = acc_sq
        acc_v[pl.ds(L, L)] = acc_m
        pltpu.sync_copy(acc_v.at[pl.ds(0, L)], out_hbm.at[pl.ds(wid * L, L)])
        pltpu.sync_copy(
            acc_v.at[pl.ds(L, L)], out_hbm.at[pl.ds(NW * L + wid * L, L)]
        )

    return kern(logits, target)


def _tc_finalize(partials):
    def body(p_ref, o_ref):
        p = p_ref[...]
        s = jnp.sum(p[0:4])
        m = jnp.sum(p[4:8])
        o_ref[...] = jnp.full((1, 1), s / m, jnp.float32)

    return pl.pallas_call(
        body,
        out_shape=jax.ShapeDtypeStruct((1, 1), jnp.float32),
    )(partials)


@jax.jit
def kernel(contrast_logits, contrast_target):
    partials = _sc_partials(contrast_logits, contrast_target)
    loss = _tc_finalize(partials.reshape(8, 128))
    return loss[0, 0]
